# baseline (speedup 1.0000x reference)
import functools

import numpy as np
import jax
import jax.numpy as jnp
from jax import lax
from jax.experimental import pallas as pl
from jax.experimental.pallas import tpu as pltpu
from jax.experimental.pallas import tpu_sc as plsc

_SHAPES = [(128, 128), (64, 64), (32, 32), (16, 16), (8, 8)]
_STRIDES = [8, 16, 32, 64, 128]
_LIMITS = [(-1.0, 64.0), (64.0, 128.0), (128.0, 256.0), (256.0, 512.0), (512.0, 999999.0)]
_BIG = 99999999.0

_B, _M, _MP = 4, 50, 64
_HW = sum(h * w for h, w in _SHAPES)
_NW = 32
_WPB = _NW // _B
_LANES = 16

_LSIZES = [h * w for h, w in _SHAPES]
_LBASES = [sum(_LSIZES[:i]) for i in range(5)]
_SEGS = [16384 // 8, 4096 // 8, 1024 // 8, 256 // 8, 16]
_VOFFS = [0, 2048, 2560, 2688, 2720]
_PER_W = 2736
_NCHUNKS = [s // _LANES for s in _SEGS]

_ENT = 56
_TROW = 5 * _ENT * _LANES


def _build_loc_table():
    xs, ys = [], []
    for (h, w), s in zip(_SHAPES, _STRIDES):
        ix = np.arange(h * w)
        xs.append((ix % w).astype(np.float32) * s + s // 2)
        ys.append((ix // w).astype(np.float32) * s + s // 2)
    return np.stack([np.concatenate(xs), np.concatenate(ys)])


_LOC_TABLE = _build_loc_table()


def _sc_body(loc_hbm, boxes_hbm, classes_hbm, cls_out, ctr_out, reg_out,
             x_v, y_v, boxes_v, classes_v, cls_ov, ctr_ov, reg_ov,
             tab_v, idx_v):
    wid = lax.axis_index("s") * 2 + lax.axis_index("c")
    batch = wid // _WPB
    part = wid % _WPB
    part4 = part % 4

    for lvl in range(5):
        seg = _SEGS[lvl]
        p = part4 if lvl == 4 else part
        src = _LBASES[lvl] + p * seg
        for row, dst in ((0, x_v), (1, y_v)):
            pltpu.sync_copy(loc_hbm.at[pl.ds(row * _HW + src, seg)],
                            dst.at[pl.ds(_VOFFS[lvl], seg)])
    pltpu.sync_copy(boxes_hbm.at[pl.ds(batch * 4 * _MP, 4 * _MP)], boxes_v)
    pltpu.sync_copy(classes_hbm.at[pl.ds(batch * _MP, _MP)], classes_v)

    counts = [jnp.int32(0)] * 5
    for g in range(4):
        gs = pl.ds(g * _LANES, _LANES)
        x1v = boxes_v[pl.ds(0 * _MP + g * _LANES, _LANES)]
        y1v = boxes_v[pl.ds(1 * _MP + g * _LANES, _LANES)]
        x2v = boxes_v[pl.ds(2 * _MP + g * _LANES, _LANES)]
        y2v = boxes_v[pl.ds(3 * _MP + g * _LANES, _LANES)]
        wv = x2v - x1v
        hv = y2v - y1v
        maxdv = jnp.maximum(wv, hv)
        cxv = (x1v + x2v) * 0.5
        cyv = (y1v + y2v) * 0.5
        areav = wv * hv
        acts = []
        for lvl in range(5):
            lo, hi = _LIMITS[lvl]
            s3 = 3.0 * _STRIDES[lvl]
            a = (maxdv <= 2.0 * hi + 2.0) & (maxdv > 2.0 * lo - s3 - 2.0)
            acts.append(jnp.where(a, 1, 0).astype(jnp.int32))
        for lane in range(_LANES):
            k = g * _LANES + lane
            if k >= _M:
                break
            rows = [jnp.broadcast_to(v[lane], (_LANES,))
                    for v in (x1v, y1v, x2v, y2v, cxv, cyv, areav)]
            kvec = jnp.full((_LANES,), k, jnp.int32)
            for lvl in range(5):
                f = acts[lvl][lane]
                n = counts[lvl]
                at = pl.ds(lvl * _ENT * _LANES + n * _LANES, _LANES)

                @pl.when(f > 0)
                def _(at=at, rows=rows, kvec=kvec):
                    for r in range(7):
                        tab_v[r, at] = rows[r]
                    idx_v[at] = kvec

                counts[lvl] = n + f
    pcounts = []
    for lvl in range(5):
        n = counts[lvl]
        for d in range(3):
            at = pl.ds(lvl * _ENT * _LANES + (n + d) * _LANES, _LANES)
            tab_v[0, at] = jnp.full((_LANES,), 1e9, jnp.float32)
            tab_v[6, at] = jnp.full((_LANES,), _BIG, jnp.float32)
            idx_v[at] = jnp.zeros((_LANES,), jnp.int32)
        pcounts.append((n + 3) // 4)

    iota = lax.iota(jnp.int32, _LANES)

    for lvl in range(5):
        lo, hi = _LIMITS[lvl]
        rad = _STRIDES[lvl] * 1.5
        lbase = lvl * _ENT * _LANES
        nquads = pcounts[lvl]

        def chunk(c, carry, lvl=lvl, lo=lo, hi=hi, rad=rad,
                  lbase=lbase, nquads=nquads):
            base = _VOFFS[lvl] + c * _LANES
            xv = x_v[pl.ds(base, _LANES)]
            yv = y_v[pl.ds(base, _LANES)]

            def quad(q, st, lbase=lbase):
                best_a, best_i = st
                for t in range(4):
                    es = pl.ds(lbase + (q * 4 + t) * _LANES, _LANES)
                    x1 = tab_v[0, es]
                    y1 = tab_v[1, es]
                    x2 = tab_v[2, es]
                    y2 = tab_v[3, es]
                    cx = tab_v[4, es]
                    cy = tab_v[5, es]
                    area = tab_v[6, es]
                    kv = idx_v[es]
                    l = xv - x1
                    tt = yv - y1
                    r = x2 - xv
                    b = y2 - yv
                    omin = jnp.minimum(jnp.minimum(l, tt), jnp.minimum(r, b))
                    omax = jnp.maximum(jnp.maximum(l, tt), jnp.maximum(r, b))
                    m_c = jnp.maximum(jnp.abs(xv - cx), jnp.abs(yv - cy)) < rad
                    mask = ((omin > 0.0) & (omax > lo) & (omax <= hi) & m_c)
                    upd = mask & (area < best_a)
                    best_a = jnp.where(upd, area, best_a)
                    best_i = jnp.where(upd, kv, best_i)
                return best_a, best_i

            best_a = jnp.full((_LANES,), _BIG, jnp.float32)
            best_i = jnp.zeros((_LANES,), jnp.int32)
            best_a, best_i = lax.fori_loop(0, nquads, quad, (best_a, best_i))

            pos = best_a < _BIG
            x1g = plsc.load_gather(boxes_v, [best_i])
            y1g = plsc.load_gather(boxes_v, [best_i + _MP])
            x2g = plsc.load_gather(boxes_v, [best_i + 2 * _MP])
            y2g = plsc.load_gather(boxes_v, [best_i + 3 * _MP])
            clsg = plsc.load_gather(classes_v, [best_i])
            lg = xv - x1g
            tg = yv - y1g
            rg = x2g - xv
            bg = y2g - yv
            lrmin = jnp.minimum(lg, rg)
            lrmax = jnp.maximum(lg, rg)
            tbmin = jnp.minimum(tg, bg)
            tbmax = jnp.maximum(tg, bg)
            num = jnp.where(pos, lrmin * tbmin, 1.0)
            den = jnp.where(pos, jnp.maximum(lrmax * tbmax + 1e-10, 0.0), 1.0)
            ratio = num / den
            bits = lax.bitcast_convert_type(ratio, jnp.int32)
            sq = lax.bitcast_convert_type(
                lax.shift_right_logical(bits, 1) + 0x1FBD1DF5, jnp.float32)
            for _ in range(3):
                sq = 0.5 * (sq + ratio / sq)

            sl = pl.ds(base, _LANES)
            cls_ov[sl] = jnp.where(pos, clsg, 0)
            ctr_ov[sl] = jnp.where(pos, sq, -1.0)
            ridx = (iota + base) * 4
            plsc.store_scatter(reg_ov, [ridx], jnp.where(pos, lg, -1.0))
            plsc.store_scatter(reg_ov, [ridx + 1], jnp.where(pos, tg, -1.0))
            plsc.store_scatter(reg_ov, [ridx + 2], jnp.where(pos, rg, -1.0))
            plsc.store_scatter(reg_ov, [ridx + 3], jnp.where(pos, bg, -1.0))
            return carry

        if lvl == 4:
            nch = jnp.where(part < 4, 1, 0)
        else:
            nch = _NCHUNKS[lvl]
        lax.fori_loop(0, nch, chunk, 0)

    for lvl in range(5):
        seg = _SEGS[lvl]
        p = part4 if lvl == 4 else part
        dst = batch * _HW + _LBASES[lvl] + p * seg
        voff = _VOFFS[lvl]

        def emit(lvl=lvl, seg=seg, dst=dst, voff=voff):
            pltpu.sync_copy(cls_ov.at[pl.ds(voff, seg)],
                            cls_out.at[pl.ds(dst, seg)])
            pltpu.sync_copy(ctr_ov.at[pl.ds(voff, seg)],
                            ctr_out.at[pl.ds(dst, seg)])
            pltpu.sync_copy(reg_ov.at[pl.ds(voff * 4, seg * 4)],
                            reg_out.at[pl.ds(dst * 4, seg * 4)])

        if lvl == 4:
            @pl.when(part < 4)
            def _():
                emit()
        else:
            emit()


@jax.jit
def _gen_targets(gt_boxes, classes):
    loc = jnp.asarray(_LOC_TABLE).reshape(-1)
    boxes_pl = jnp.transpose(gt_boxes, (0, 2, 1))
    boxes_pl = jnp.pad(boxes_pl, ((0, 0), (0, 0), (0, _MP - _M))).reshape(-1)
    classes_p = jnp.pad(classes, ((0, 0), (0, _MP - _M))).reshape(-1)

    mesh = plsc.VectorSubcoreMesh(core_axis_name="c", subcore_axis_name="s")
    run = functools.partial(
        pl.kernel,
        mesh=mesh,
        compiler_params=pltpu.CompilerParams(
            needs_layout_passes=False, use_tc_tiling_on_sc=False),
        out_type=[
            jax.ShapeDtypeStruct((_B * _HW,), jnp.int32),
            jax.ShapeDtypeStruct((_B * _HW,), jnp.float32),
            jax.ShapeDtypeStruct((_B * _HW * 4,), jnp.float32),
        ],
        scratch_types=[
            pltpu.VMEM((_PER_W,), jnp.float32),
            pltpu.VMEM((_PER_W,), jnp.float32),
            pltpu.VMEM((4 * _MP,), jnp.float32),
            pltpu.VMEM((_MP,), jnp.int32),
            pltpu.VMEM((_PER_W,), jnp.int32),
            pltpu.VMEM((_PER_W,), jnp.float32),
            pltpu.VMEM((_PER_W * 4,), jnp.float32),
            pltpu.VMEM((7, _TROW), jnp.float32),
            pltpu.VMEM((_TROW,), jnp.int32),
        ],
    )(_sc_body)
    cls_p, ctr_p, reg_p = run(loc, boxes_pl, classes_p)
    return (cls_p.reshape(_B, _HW, 1), ctr_p.reshape(_B, _HW, 1),
            reg_p.reshape(_B, _HW, 4))


def kernel(cls_logits_0, cls_logits_1, cls_logits_2, cls_logits_3, cls_logits_4,
           ctr_logits_0, ctr_logits_1, ctr_logits_2, ctr_logits_3, ctr_logits_4,
           reg_preds_0, reg_preds_1, reg_preds_2, reg_preds_3, reg_preds_4,
           gt_boxes, classes):
    return _gen_targets(gt_boxes, classes)
